# baseline (speedup 1.0000x reference)
import functools

import jax
import jax.numpy as jnp
from jax import lax
from jax.experimental import pallas as pl
from jax.experimental.pallas import tpu as pltpu
from jax.experimental.pallas import tpu_sc as plsc

NUM_WORKERS = 32
LANES = 16
CHUNK = 16384
TAB = 128
PAD = 128

I_LO = -1.5
I_HI = 1.5
NUM_POINTS = 100
INV_H = (NUM_POINTS - 1) / (I_HI - I_LO)
OFF = -I_LO * INV_H
MAX_BUCKET = NUM_POINTS - 2


@functools.lru_cache(maxsize=None)
def _make_fhn_kernel(n: int):
    per_worker = n // NUM_WORKERS
    nch = per_worker // CHUNK
    mesh = plsc.VectorSubcoreMesh(core_axis_name="c", subcore_axis_name="s")

    @functools.partial(
        pl.kernel,
        mesh=mesh,
        out_type=jax.ShapeDtypeStruct((n,), jnp.float32),
        compiler_params=pltpu.CompilerParams(needs_layout_passes=False),
        scratch_types=[
            pltpu.VMEM((PAD,), jnp.float32),
            pltpu.VMEM((PAD,), jnp.float32),
            pltpu.VMEM((TAB,), jnp.float32),
            pltpu.VMEM((TAB,), jnp.float32),
            pltpu.VMEM((CHUNK,), jnp.float32),
            pltpu.VMEM((CHUNK,), jnp.float32),
            pltpu.VMEM((CHUNK,), jnp.float32),
            pltpu.VMEM((CHUNK,), jnp.float32),
            pltpu.SemaphoreType.DMA,
            pltpu.SemaphoreType.DMA,
            pltpu.SemaphoreType.DMA,
            pltpu.SemaphoreType.DMA,
        ],
    )
    def fhn(x_hbm, ipad_hbm, vpad_hbm, out_hbm, ivm, vvm, atab, btab,
            xb0, xb1, yb0, yb1, si0, si1, so0, so1):
        cid = lax.axis_index("c")
        sid = lax.axis_index("s")
        wid = sid * 2 + cid
        base = wid * per_worker

        pltpu.sync_copy(ipad_hbm, ivm)
        pltpu.sync_copy(vpad_hbm, vvm)

        lane = lax.iota(jnp.int32, LANES)
        for k in range(TAB // LANES):
            ii = lane + (k * LANES)
            ii1 = jnp.minimum(ii + 1, PAD - 1)
            i0 = plsc.load_gather(ivm, [ii])
            i1 = plsc.load_gather(ivm, [ii1])
            v0 = plsc.load_gather(vvm, [ii])
            v1 = plsc.load_gather(vvm, [ii1])
            slope = (v1 - v0) / (i1 - i0)
            atab[pl.ds(k * LANES, LANES)] = v0 - slope * i0
            btab[pl.ds(k * LANES, LANES)] = slope

        def in_copy(ch, buf, sem):
            off = base + ch * CHUNK
            return pltpu.make_async_copy(x_hbm.at[pl.ds(off, CHUNK)], buf, sem)

        def out_copy(ch, buf, sem):
            off = base + ch * CHUNK
            return pltpu.make_async_copy(buf, out_hbm.at[pl.ds(off, CHUNK)], sem)

        def compute(xbuf, ybuf):
            @plsc.parallel_loop(0, CHUNK // LANES, unroll=16)
            def _(i):
                o = i * LANES
                xv = xbuf[pl.ds(o, LANES)]
                xc = jnp.minimum(jnp.maximum(xv, I_LO), I_HI)
                idx = (xc * INV_H + OFF).astype(jnp.int32)
                idx = jnp.minimum(idx, MAX_BUCKET)
                a = plsc.load_gather(atab, [idx])
                b = plsc.load_gather(btab, [idx])
                ybuf[pl.ds(o, LANES)] = a + b * xc

        xbufs, ybufs = (xb0, xb1), (yb0, yb1)
        sis, sos = (si0, si1), (so0, so1)
        in_copy(0, xb0, si0).start()
        in_copy(1, xb1, si1).start()
        for ch in range(nch):
            p = ch & 1
            in_copy(ch, xbufs[p], sis[p]).wait()
            if ch >= 2:
                out_copy(ch - 2, ybufs[p], sos[p]).wait()
            compute(xbufs[p], ybufs[p])
            out_copy(ch, ybufs[p], sos[p]).start()
            if ch + 2 < nch:
                in_copy(ch + 2, xbufs[p], sis[p]).start()
        out_copy(nch - 2, ybufs[nch & 1], sos[nch & 1]).wait()
        out_copy(nch - 1, ybufs[(nch + 1) & 1], sos[(nch + 1) & 1]).wait()

    return fhn


def kernel(x, I_ext_values, v_values):
    npts = I_ext_values.shape[0]
    pad = PAD - npts
    step = I_ext_values[-1] - I_ext_values[-2]
    ipad = jnp.concatenate(
        [I_ext_values,
         I_ext_values[-1] + step * jnp.arange(1, pad + 1, dtype=jnp.float32)])
    vpad = jnp.concatenate(
        [v_values, jnp.broadcast_to(v_values[-1], (pad,))])
    return _make_fhn_kernel(x.shape[0])(x, ipad, vpad)


# speedup vs baseline: 1.2416x; 1.2416x over previous
import functools

import jax
import jax.numpy as jnp
from jax import lax
from jax.experimental import pallas as pl
from jax.experimental.pallas import tpu as pltpu
from jax.experimental.pallas import tpu_sc as plsc

NUM_WORKERS = 32
LANES = 16
CHUNK = 16384
TAB = 128
PAD = 128

I_LO = -1.5
I_HI = 1.5
NUM_POINTS = 100
INV_H = (NUM_POINTS - 1) / (I_HI - I_LO)
OFF = -I_LO * INV_H
MAX_BUCKET = NUM_POINTS - 2


@functools.lru_cache(maxsize=None)
def _make_fhn_kernel(n: int):
    per_worker = n // NUM_WORKERS
    nch = per_worker // CHUNK
    mesh = plsc.VectorSubcoreMesh(core_axis_name="c", subcore_axis_name="s")

    @functools.partial(
        pl.kernel,
        mesh=mesh,
        out_type=jax.ShapeDtypeStruct((n,), jnp.float32),
        compiler_params=pltpu.CompilerParams(needs_layout_passes=False),
        scratch_types=[
            pltpu.VMEM((PAD,), jnp.float32),
            pltpu.VMEM((PAD,), jnp.float32),
            pltpu.VMEM((TAB,), jnp.float32),
            pltpu.VMEM((TAB,), jnp.float32),
            pltpu.VMEM((CHUNK,), jnp.float32),
            pltpu.VMEM((CHUNK,), jnp.float32),
            pltpu.VMEM((CHUNK,), jnp.float32),
            pltpu.VMEM((CHUNK,), jnp.float32),
            pltpu.SemaphoreType.DMA,
            pltpu.SemaphoreType.DMA,
            pltpu.SemaphoreType.DMA,
            pltpu.SemaphoreType.DMA,
        ],
    )
    def fhn(x_hbm, ipad_hbm, vpad_hbm, out_hbm, ivm, vvm, atab, btab,
            xb0, xb1, yb0, yb1, si0, si1, so0, so1):
        cid = lax.axis_index("c")
        sid = lax.axis_index("s")
        wid = sid * 2 + cid
        base = wid * per_worker

        pltpu.sync_copy(ipad_hbm, ivm)
        pltpu.sync_copy(vpad_hbm, vvm)

        lane = lax.iota(jnp.int32, LANES)
        for k in range(TAB // LANES):
            ii = lane + (k * LANES)
            ii1 = jnp.minimum(ii + 1, PAD - 1)
            i0 = plsc.load_gather(ivm, [ii])
            i1 = plsc.load_gather(ivm, [ii1])
            v0 = plsc.load_gather(vvm, [ii])
            v1 = plsc.load_gather(vvm, [ii1])
            slope = (v1 - v0) / (i1 - i0)
            atab[pl.ds(k * LANES, LANES)] = v0 - slope * i0
            btab[pl.ds(k * LANES, LANES)] = slope

        def in_copy(ch, buf, sem):
            off = base + ch * CHUNK
            return pltpu.make_async_copy(x_hbm.at[pl.ds(off, CHUNK)], buf, sem)

        def out_copy(ch, buf, sem):
            off = base + ch * CHUNK
            return pltpu.make_async_copy(buf, out_hbm.at[pl.ds(off, CHUNK)], sem)

        def compute(xbuf, ybuf):
            @plsc.parallel_loop(0, CHUNK // LANES, unroll=8)
            def _(i):
                o = i * LANES
                xv = xbuf[pl.ds(o, LANES)]
                xc = jnp.minimum(jnp.maximum(xv, I_LO), I_HI)
                idx = (xc * INV_H + OFF).astype(jnp.int32)
                idx = jnp.minimum(idx, MAX_BUCKET)
                a = plsc.load_gather(atab, [idx])
                b = plsc.load_gather(btab, [idx])
                ybuf[pl.ds(o, LANES)] = a + b * xc

        xbufs, ybufs = (xb0, xb1), (yb0, yb1)
        sis, sos = (si0, si1), (so0, so1)
        npairs = nch // 2
        in_copy(0, xb0, si0).start()
        in_copy(1, xb1, si1).start()

        def pair_body(g, carry):
            for p in (0, 1):
                ch = 2 * g + p
                in_copy(ch, xbufs[p], sis[p]).wait()

                @pl.when(g >= 1)
                def _wait_prev_out():
                    out_copy(ch - 2, ybufs[p], sos[p]).wait()

                compute(xbufs[p], ybufs[p])
                out_copy(ch, ybufs[p], sos[p]).start()

                @pl.when(g < npairs - 1)
                def _start_next_in():
                    in_copy(ch + 2, xbufs[p], sis[p]).start()

            return carry

        lax.fori_loop(0, npairs, pair_body, 0)
        out_copy(nch - 2, yb0, so0).wait()
        out_copy(nch - 1, yb1, so1).wait()

    return fhn


def kernel(x, I_ext_values, v_values):
    npts = I_ext_values.shape[0]
    pad = PAD - npts
    step = I_ext_values[-1] - I_ext_values[-2]
    ipad = jnp.concatenate(
        [I_ext_values,
         I_ext_values[-1] + step * jnp.arange(1, pad + 1, dtype=jnp.float32)])
    vpad = jnp.concatenate(
        [v_values, jnp.broadcast_to(v_values[-1], (pad,))])
    return _make_fhn_kernel(x.shape[0])(x, ipad, vpad)
